# baseline (speedup 1.0000x reference)
import functools

import jax
import jax.numpy as jnp
from jax import lax
from jax.experimental import pallas as pl
from jax.experimental.pallas import tpu as pltpu
from jax.experimental.pallas import tpu_sc as plsc

NA = 4096
NB = 4
GRID = 8
NGRID = GRID ** 3
NG = NB * NGRID
NT = 16
CODE = 32
H = 128
NL = 4
KA = 8
KG = 24
CUTOFF = 8.0
SPACING = 1.5

_C0 = 0.28209479177387814
_C1 = 0.4886025119029199
_C2 = 1.0925484305920792
_C2M0 = 0.31539156525252005
_C2P2 = 0.5462742152960396

EDGES = NA * KA + NG * KG
_SC_WORKERS = 32
_ROWS_PER_GATHER = 128
_GATHERS_PER_WORKER = EDGES // (_SC_WORKERS * _ROWS_PER_GATHER)


def _knn_body(K, diag, RB, C, dstT_ref, dstbT_ref, pos8_ref, bat8_ref,
              nbrT_ref):
    b = pl.program_id(0)
    pxc = pos8_ref[:, 0:1]
    pyc = pos8_ref[:, 1:2]
    pzc = pos8_ref[:, 2:3]
    batc = bat8_ref[:, 0:1]
    dx = pxc - dstT_ref[0:1, :]
    dy = pyc - dstT_ref[1:2, :]
    dz = pzc - dstT_ref[2:3, :]
    d2 = dx * dx + dy * dy + dz * dz
    inf = jnp.float32(jnp.inf)
    same = batc == dstbT_ref[0:1, :]
    d2 = jnp.where(same, d2, inf)
    rows = lax.broadcasted_iota(jnp.int32, (C, RB), 0)
    if diag:
        cols = lax.broadcasted_iota(jnp.int32, (C, RB), 1) + b * RB
        d2 = jnp.where(rows == cols, inf, d2)

    def body(k, d2):
        mn = jnp.min(d2, axis=0, keepdims=True)
        idx = jnp.min(jnp.where(d2 == mn, rows, C), axis=0, keepdims=True)
        nbrT_ref[pl.ds(k, 1), :] = idx
        return jnp.where(rows == idx, inf, d2)

    lax.fori_loop(0, K, body, d2)


def _knn_call(dstT, dstbT, pos8, bat8, K, diag, RB, interpret=False):
    n = dstT.shape[1]
    C = pos8.shape[0]
    grid = (n // RB,)
    body = functools.partial(_knn_body, K, diag, RB, C)
    return pl.pallas_call(
        body,
        grid=grid,
        in_specs=[
            pl.BlockSpec((8, RB), lambda i: (0, i)),
            pl.BlockSpec((8, RB), lambda i: (0, i)),
            pl.BlockSpec((C, 8), lambda i: (0, 0)),
            pl.BlockSpec((C, 8), lambda i: (0, 0)),
        ],
        out_specs=pl.BlockSpec((K, RB), lambda i: (0, i)),
        out_shape=jax.ShapeDtypeStruct((K, n), jnp.int32),
        interpret=interpret,
    )(dstT, dstbT, pos8, bat8)


def _eattr_body(K, RB, gp3_ref, dst8_ref, sh3_ref, attr_ref):
    gx = gp3_ref[:, :, 0]
    gy = gp3_ref[:, :, 1]
    gz = gp3_ref[:, :, 2]
    rx = gx - dst8_ref[:, 0:1]
    ry = gy - dst8_ref[:, 1:2]
    rz = gz - dst8_ref[:, 2:3]
    ex = rx + 1e-12
    ey = ry + 1e-12
    ez = rz + 1e-12
    dist = jnp.sqrt(ex * ex + ey * ey + ez * ez)
    valid = (dist <= CUTOFF).astype(jnp.float32)
    rn = jnp.sqrt(rx * rx + ry * ry + rz * rz)
    inv = 1.0 / (rn + 1e-12)
    x = rx * inv
    y = ry * inv
    z = rz * inv
    comps = [
        jnp.full_like(x, _C0),
        _C1 * y, _C1 * z, _C1 * x,
        _C2 * x * y, _C2 * y * z, _C2M0 * (3.0 * z * z - 1.0),
        _C2 * x * z, _C2P2 * (x * x - y * y),
    ]
    sums = []
    for c in range(9):
        v = comps[c] * valid
        sh3_ref[:, :, c] = v
        sums.append(jnp.sum(v, axis=1, keepdims=True))
    sh3_ref[:, :, 9] = valid
    for c in range(10, 16):
        sh3_ref[:, :, c] = jnp.zeros((RB, K), jnp.float32)
    cnt = jnp.sum(valid, axis=1, keepdims=True)
    denom = jnp.maximum(cnt, 1.0)
    attr_ref[...] = jnp.concatenate(
        [jnp.ones((RB, 1), jnp.float32)]
        + [sums[c] / denom for c in range(1, 9)]
        + [jnp.zeros((RB, 7), jnp.float32)], axis=1)


def _eattr_call(gp3, dst8, K, RB, interpret=False):
    n = dst8.shape[0]
    body = functools.partial(_eattr_body, K, RB)
    return pl.pallas_call(
        body,
        grid=(n // RB,),
        in_specs=[
            pl.BlockSpec((RB, K, 16), lambda i: (i, 0, 0)),
            pl.BlockSpec((RB, 8), lambda i: (i, 0)),
        ],
        out_specs=[
            pl.BlockSpec((RB, K, 16), lambda i: (i, 0, 0)),
            pl.BlockSpec((RB, 16), lambda i: (i, 0)),
        ],
        out_shape=[
            jax.ShapeDtypeStruct((n, K, 16), jnp.float32),
            jax.ShapeDtypeStruct((n, 16), jnp.float32),
        ],
        interpret=interpret,
    )(gp3, dst8)


def _embed_body(types_ref, wemb_ref, wms_ref, out_ref, p1_ref):
    t = types_ref[:, 0:1]
    ids = lax.broadcasted_iota(jnp.int32, (1, NT), 1)
    oh = (t == ids).astype(jnp.float32)
    h = jnp.dot(oh, wemb_ref[...], preferred_element_type=jnp.float32)
    out_ref[...] = h
    p1_ref[...] = jnp.dot(h, wms_ref[...], preferred_element_type=jnp.float32)


def _embed_call(types8, W_embed, wm_s0, RB=512, interpret=False):
    return pl.pallas_call(
        _embed_body,
        grid=(NA // RB,),
        in_specs=[
            pl.BlockSpec((RB, 8), lambda i: (i, 0)),
            pl.BlockSpec((NT, H), lambda i: (0, 0)),
            pl.BlockSpec((H, H), lambda i: (0, 0)),
        ],
        out_specs=[
            pl.BlockSpec((RB, H), lambda i: (i, 0)),
            pl.BlockSpec((RB, H), lambda i: (i, 0)),
        ],
        out_shape=[
            jax.ShapeDtypeStruct((NA, H), jnp.float32),
            jax.ShapeDtypeStruct((NA, H), jnp.float32),
        ],
        interpret=interpret,
    )(types8, W_embed, wm_s0)


def _msg_body(K, RB, want_p1, *refs):
    if want_p1:
        (g3_ref, sh3_ref, h_ref, attr_ref, wmd_ref, we_ref,
         wuh_ref, wua_ref, wun_ref, wmsn_ref, out_ref, p1_ref) = refs
    else:
        (g3_ref, sh3_ref, h_ref, attr_ref, wmd_ref, we_ref,
         wuh_ref, wua_ref, wun_ref, out_ref) = refs
    hb = h_ref[...]
    p2 = jnp.dot(hb, wmd_ref[...], preferred_element_type=jnp.float32)
    agg = jnp.zeros((RB, H), jnp.float32)
    for k in range(K):
        shk = sh3_ref[:, k, :]
        ep = jnp.dot(shk, we_ref[...], preferred_element_type=jnp.float32)
        gk = g3_ref[:, k, :]
        mk = jax.nn.silu(gk + p2 + ep) * shk[:, 9:10]
        agg = agg + mk
    u = (jnp.dot(hb, wuh_ref[...], preferred_element_type=jnp.float32)
         + jnp.dot(agg, wua_ref[...], preferred_element_type=jnp.float32)
         + jnp.dot(attr_ref[...], wun_ref[...],
                   preferred_element_type=jnp.float32))
    hn = hb + jax.nn.silu(u)
    out_ref[...] = hn
    if want_p1:
        p1_ref[...] = jnp.dot(hn, wmsn_ref[...],
                              preferred_element_type=jnp.float32)


def _msg_call(g3, sh3, h, attr, wmd, we, wuh, wua, wun, K, RB,
              wms_next=None, interpret=False):
    n = h.shape[0]
    want_p1 = wms_next is not None
    body = functools.partial(_msg_body, K, RB, want_p1)
    in_specs = [
        pl.BlockSpec((RB, K, H), lambda i: (i, 0, 0)),
        pl.BlockSpec((RB, K, 16), lambda i: (i, 0, 0)),
        pl.BlockSpec((RB, H), lambda i: (i, 0)),
        pl.BlockSpec((RB, 16), lambda i: (i, 0)),
        pl.BlockSpec((H, H), lambda i: (0, 0)),
        pl.BlockSpec((16, H), lambda i: (0, 0)),
        pl.BlockSpec((H, H), lambda i: (0, 0)),
        pl.BlockSpec((H, H), lambda i: (0, 0)),
        pl.BlockSpec((16, H), lambda i: (0, 0)),
    ]
    args = [g3, sh3, h, attr, wmd, we, wuh, wua, wun]
    out_specs = pl.BlockSpec((RB, H), lambda i: (i, 0))
    out_shape = jax.ShapeDtypeStruct((n, H), jnp.float32)
    if want_p1:
        in_specs.append(pl.BlockSpec((H, H), lambda i: (0, 0)))
        args.append(wms_next)
        out_specs = [out_specs, pl.BlockSpec((RB, H), lambda i: (i, 0))]
        out_shape = [out_shape, jax.ShapeDtypeStruct((n, H), jnp.float32)]
    return pl.pallas_call(
        body,
        grid=(n // RB,),
        in_specs=in_specs,
        out_specs=out_specs,
        out_shape=out_shape,
        interpret=interpret,
    )(*args)


def _out_body(h_ref, w_ref, out_ref):
    out_ref[...] = jnp.dot(h_ref[...], w_ref[...],
                           preferred_element_type=jnp.float32)


def _out_call(h_g, W_out, RB=512, interpret=False):
    return pl.pallas_call(
        _out_body,
        grid=(NG // RB,),
        in_specs=[
            pl.BlockSpec((RB, H), lambda i: (i, 0)),
            pl.BlockSpec((H, CODE), lambda i: (0, 0)),
        ],
        out_specs=pl.BlockSpec((RB, CODE), lambda i: (i, 0)),
        out_shape=jax.ShapeDtypeStruct((NG, CODE), jnp.float32),
        interpret=interpret,
    )(h_g, W_out)


_AA_CHUNKS = NA * KA // _ROWS_PER_GATHER


def _sc_gather_body(src_hbm, table_hbm, outa_hbm, outg_hbm, idx_v, buf, sem):
    c = lax.axis_index("c")
    s = lax.axis_index("s")
    wid = s * 2 + c
    nblk = _GATHERS_PER_WORKER
    pltpu.sync_copy(src_hbm.at[wid], idx_v)
    for j in range(nblk):
        pltpu.async_copy(table_hbm.at[idx_v.at[j]], buf, sem).wait()
        chunk = wid + _SC_WORKERS * j
        if j < _AA_CHUNKS // _SC_WORKERS:
            pltpu.sync_copy(
                buf, outa_hbm.at[pl.ds(chunk * _ROWS_PER_GATHER,
                                       _ROWS_PER_GATHER)])
        else:
            pltpu.sync_copy(
                buf, outg_hbm.at[pl.ds((chunk - _AA_CHUNKS) * _ROWS_PER_GATHER,
                                       _ROWS_PER_GATHER)])


def _sc_gather(src3, table):
    D = table.shape[1]
    mesh = plsc.VectorSubcoreMesh(core_axis_name="c", subcore_axis_name="s")
    params = (pltpu.CompilerParams(use_tc_tiling_on_sc=False)
              if D < 128 else None)
    fn = pl.kernel(
        _sc_gather_body,
        out_type=[
            jax.ShapeDtypeStruct((NA * KA, D), jnp.float32),
            jax.ShapeDtypeStruct((NG * KG, D), jnp.float32),
        ],
        mesh=mesh,
        compiler_params=params,
        scratch_types=[
            pltpu.VMEM((_GATHERS_PER_WORKER, _ROWS_PER_GATHER), jnp.int32),
            pltpu.VMEM((_ROWS_PER_GATHER, D), jnp.float32),
            pltpu.SemaphoreType.DMA,
        ],
    )
    return fn(src3, table)


def _grid_xyz():
    lin = (jnp.arange(GRID, dtype=jnp.float32) - (GRID - 1) / 2.0) * SPACING
    gx, gy, gz = jnp.meshgrid(lin, lin, lin, indexing='ij')
    return jnp.stack([gx.ravel(), gy.ravel(), gz.ravel()], axis=-1)


def _pipeline(pos, atom_types, batch, W_embed, W_msg, W_upd, W_out,
              gather_fn, interpret=False):
    f32 = jnp.float32
    i32 = jnp.int32
    batch = batch.astype(i32)
    atom_types = atom_types.astype(i32)

    grid_flat = jnp.tile(_grid_xyz(), (NB, 1))
    grid_batch = jnp.repeat(jnp.arange(NB, dtype=i32), NGRID)

    pos8 = jnp.pad(pos, ((0, 0), (0, 5)))
    posT = jnp.pad(pos.T, ((0, 5), (0, 0)))
    batT = jnp.broadcast_to(batch[None, :], (8, NA))
    bat8 = jnp.broadcast_to(batch[:, None], (NA, 8))
    gridT = jnp.pad(grid_flat.T, ((0, 5), (0, 0)))
    grid8 = jnp.pad(grid_flat, ((0, 0), (0, 5)))
    batGT = jnp.broadcast_to(grid_batch[None, :], (8, NG))
    types8 = jnp.broadcast_to(atom_types[:, None], (NA, 8))

    nbrTA = _knn_call(posT, batT, pos8, bat8, KA, True, 256,
                      interpret=interpret)
    nbrTG = _knn_call(gridT, batGT, pos8, bat8, KG, False, 256,
                      interpret=interpret)
    nbrA = nbrTA.T
    nbrG = nbrTG.T

    src_chunks = jnp.concatenate(
        [nbrA.reshape(-1), nbrG.reshape(-1)]).reshape(
            _GATHERS_PER_WORKER, _SC_WORKERS, _ROWS_PER_GATHER)
    src3 = src_chunks.transpose(1, 0, 2)

    pos16 = jnp.pad(pos, ((0, 0), (0, 13)))
    GPa, GPg = gather_fn(src3, pos16)
    shA, attrA = _eattr_call(GPa.reshape(NA, KA, 16), pos8, KA, 256,
                             interpret=interpret)
    shG, attrG = _eattr_call(GPg.reshape(NG, KG, 16), grid8, KG, 128,
                             interpret=interpret)

    h_a, P1 = _embed_call(types8, W_embed, W_msg[0, :H], interpret=interpret)
    h_g = jnp.zeros((NG, H), f32)

    for l in range(NL):
        wm_d = W_msg[l, H:2 * H]
        we = jnp.zeros((16, H), f32).at[:9].set(W_msg[l, 2 * H:])
        wu_h = W_upd[l, :H]
        wu_a = W_upd[l, H:2 * H]
        wu_n = jnp.zeros((16, H), f32).at[:9].set(W_upd[l, 2 * H:])

        GA, GG = gather_fn(src3, P1)
        Ga = GA.reshape(NA, KA, H)
        Gg = GG.reshape(NG, KG, H)
        h_a, P1 = _msg_call(Ga, shA, h_a, attrA, wm_d, we, wu_h, wu_a, wu_n,
                            KA, 512, wms_next=W_msg[(l + 1) % NL, :H],
                            interpret=interpret)
        h_g = _msg_call(Gg, shG, h_g, attrG, wm_d, we, wu_h, wu_a, wu_n,
                        KG, 512, interpret=interpret)

    out = _out_call(h_g, W_out, interpret=interpret)
    return out.reshape(NB, NGRID, CODE)


def kernel(pos, atom_types, batch, W_embed, W_msg, W_upd, W_out):
    return _pipeline(pos, atom_types, batch, W_embed, W_msg, W_upd, W_out,
                     _sc_gather, interpret=False)


# speedup vs baseline: 1.0558x; 1.0558x over previous
import functools

import jax
import jax.numpy as jnp
from jax import lax
from jax.experimental import pallas as pl
from jax.experimental.pallas import tpu as pltpu
from jax.experimental.pallas import tpu_sc as plsc

NA = 4096
NB = 4
GRID = 8
NGRID = GRID ** 3
NG = NB * NGRID
NT = 16
CODE = 32
H = 128
NL = 4
KA = 8
KG = 24
CUTOFF = 8.0
SPACING = 1.5

_C0 = 0.28209479177387814
_C1 = 0.4886025119029199
_C2 = 1.0925484305920792
_C2M0 = 0.31539156525252005
_C2P2 = 0.5462742152960396

EDGES = NA * KA + NG * KG
_SC_WORKERS = 32
_ROWS_PER_GATHER = 128
_GATHERS_PER_WORKER = EDGES // (_SC_WORKERS * _ROWS_PER_GATHER)


def _knn_body(K, diag, RB, C, dstT_ref, dstbT_ref, pos8_ref, bat8_ref,
              posT_ref, nbrT_ref, shT_ref, attrT_ref):
    b = pl.program_id(0)
    pxc = pos8_ref[:, 0:1]
    pyc = pos8_ref[:, 1:2]
    pzc = pos8_ref[:, 2:3]
    batc = bat8_ref[:, 0:1]
    dstx = dstT_ref[0:1, :]
    dsty = dstT_ref[1:2, :]
    dstz = dstT_ref[2:3, :]
    dx = pxc - dstx
    dy = pyc - dsty
    dz = pzc - dstz
    d2 = dx * dx + dy * dy + dz * dz
    inf = jnp.float32(jnp.inf)
    same = batc == dstbT_ref[0:1, :]
    d2 = jnp.where(same, d2, inf)
    rows = lax.broadcasted_iota(jnp.int32, (C, RB), 0)
    if diag:
        cols = lax.broadcasted_iota(jnp.int32, (C, RB), 1) + b * RB
        d2 = jnp.where(rows == cols, inf, d2)
    posT = posT_ref[...]

    def body(k, carry):
        d2, sums, cnt = carry
        mn = jnp.min(d2, axis=0, keepdims=True)
        idx = jnp.min(jnp.where(d2 == mn, rows, C), axis=0, keepdims=True)
        oh = rows == idx
        d2 = jnp.where(oh, inf, d2)
        nbrT_ref[pl.ds(k, 1), :] = idx
        selp = jnp.dot(posT, oh.astype(jnp.float32),
                       preferred_element_type=jnp.float32)
        rx = selp[0:1, :] - dstx
        ry = selp[1:2, :] - dsty
        rz = selp[2:3, :] - dstz
        ex = rx + 1e-12
        ey = ry + 1e-12
        ez = rz + 1e-12
        dist = jnp.sqrt(ex * ex + ey * ey + ez * ez)
        valid = (dist <= CUTOFF).astype(jnp.float32)
        rn = jnp.sqrt(rx * rx + ry * ry + rz * rz)
        inv = 1.0 / (rn + 1e-12)
        x = rx * inv
        y = ry * inv
        z = rz * inv
        sh = jnp.concatenate([
            jnp.full_like(x, _C0),
            _C1 * y, _C1 * z, _C1 * x,
            _C2 * x * y, _C2 * y * z, _C2M0 * (3.0 * z * z - 1.0),
            _C2 * x * z, _C2P2 * (x * x - y * y),
        ], axis=0) * valid
        shv = jnp.concatenate(
            [sh, valid, jnp.zeros((6, RB), jnp.float32)], axis=0)
        shT_ref[pl.ds(k, 1), :, :] = shv[None]
        return d2, sums + shv, cnt + valid

    carry = (d2, jnp.zeros((16, RB), jnp.float32),
             jnp.zeros((1, RB), jnp.float32))
    d2, sums, cnt = lax.fori_loop(0, K, body, carry)
    mean = sums / jnp.maximum(cnt, 1.0)
    attrT_ref[...] = jnp.concatenate(
        [jnp.ones((1, RB), jnp.float32), mean[1:9, :],
         jnp.zeros((7, RB), jnp.float32)], axis=0)


def _knn_call(dstT, dstbT, pos8, bat8, posT, K, diag, RB, interpret=False):
    n = dstT.shape[1]
    C = pos8.shape[0]
    grid = (n // RB,)
    body = functools.partial(_knn_body, K, diag, RB, C)
    return pl.pallas_call(
        body,
        grid=grid,
        in_specs=[
            pl.BlockSpec((8, RB), lambda i: (0, i)),
            pl.BlockSpec((8, RB), lambda i: (0, i)),
            pl.BlockSpec((C, 8), lambda i: (0, 0)),
            pl.BlockSpec((C, 8), lambda i: (0, 0)),
            pl.BlockSpec((8, C), lambda i: (0, 0)),
        ],
        out_specs=[
            pl.BlockSpec((K, RB), lambda i: (0, i)),
            pl.BlockSpec((K, 16, RB), lambda i: (0, 0, i)),
            pl.BlockSpec((16, RB), lambda i: (0, i)),
        ],
        out_shape=[
            jax.ShapeDtypeStruct((K, n), jnp.int32),
            jax.ShapeDtypeStruct((K, 16, n), jnp.float32),
            jax.ShapeDtypeStruct((16, n), jnp.float32),
        ],
        interpret=interpret,
    )(dstT, dstbT, pos8, bat8, posT)


def _embed_body(types_ref, wemb_ref, wms_ref, out_ref, p1_ref):
    t = types_ref[:, 0:1]
    ids = lax.broadcasted_iota(jnp.int32, (1, NT), 1)
    oh = (t == ids).astype(jnp.float32)
    h = jnp.dot(oh, wemb_ref[...], preferred_element_type=jnp.float32)
    out_ref[...] = h
    p1_ref[...] = jnp.dot(h, wms_ref[...], preferred_element_type=jnp.float32)


def _embed_call(types8, W_embed, wm_s0, RB=512, interpret=False):
    return pl.pallas_call(
        _embed_body,
        grid=(NA // RB,),
        in_specs=[
            pl.BlockSpec((RB, 8), lambda i: (i, 0)),
            pl.BlockSpec((NT, H), lambda i: (0, 0)),
            pl.BlockSpec((H, H), lambda i: (0, 0)),
        ],
        out_specs=[
            pl.BlockSpec((RB, H), lambda i: (i, 0)),
            pl.BlockSpec((RB, H), lambda i: (i, 0)),
        ],
        out_shape=[
            jax.ShapeDtypeStruct((NA, H), jnp.float32),
            jax.ShapeDtypeStruct((NA, H), jnp.float32),
        ],
        interpret=interpret,
    )(types8, W_embed, wm_s0)


def _msg_body(K, RB, want_p1, *refs):
    if want_p1:
        (g3_ref, sh3_ref, h_ref, attr_ref, wmd_ref, we_ref,
         wuh_ref, wua_ref, wun_ref, wmsn_ref, out_ref, p1_ref) = refs
    else:
        (g3_ref, sh3_ref, h_ref, attr_ref, wmd_ref, we_ref,
         wuh_ref, wua_ref, wun_ref, out_ref) = refs
    hb = h_ref[...]
    p2 = jnp.dot(hb, wmd_ref[...], preferred_element_type=jnp.float32)
    agg = jnp.zeros((RB, H), jnp.float32)
    for k in range(K):
        shk = sh3_ref[:, k, :]
        ep = jnp.dot(shk, we_ref[...], preferred_element_type=jnp.float32)
        gk = g3_ref[:, k, :]
        mk = jax.nn.silu(gk + p2 + ep) * shk[:, 9:10]
        agg = agg + mk
    u = (jnp.dot(hb, wuh_ref[...], preferred_element_type=jnp.float32)
         + jnp.dot(agg, wua_ref[...], preferred_element_type=jnp.float32)
         + jnp.dot(attr_ref[...], wun_ref[...],
                   preferred_element_type=jnp.float32))
    hn = hb + jax.nn.silu(u)
    out_ref[...] = hn
    if want_p1:
        p1_ref[...] = jnp.dot(hn, wmsn_ref[...],
                              preferred_element_type=jnp.float32)


def _msg_call(g3, sh3, h, attr, wmd, we, wuh, wua, wun, K, RB,
              wms_next=None, interpret=False):
    n = h.shape[0]
    want_p1 = wms_next is not None
    body = functools.partial(_msg_body, K, RB, want_p1)
    in_specs = [
        pl.BlockSpec((RB, K, H), lambda i: (i, 0, 0)),
        pl.BlockSpec((RB, K, 16), lambda i: (i, 0, 0)),
        pl.BlockSpec((RB, H), lambda i: (i, 0)),
        pl.BlockSpec((RB, 16), lambda i: (i, 0)),
        pl.BlockSpec((H, H), lambda i: (0, 0)),
        pl.BlockSpec((16, H), lambda i: (0, 0)),
        pl.BlockSpec((H, H), lambda i: (0, 0)),
        pl.BlockSpec((H, H), lambda i: (0, 0)),
        pl.BlockSpec((16, H), lambda i: (0, 0)),
    ]
    args = [g3, sh3, h, attr, wmd, we, wuh, wua, wun]
    out_specs = pl.BlockSpec((RB, H), lambda i: (i, 0))
    out_shape = jax.ShapeDtypeStruct((n, H), jnp.float32)
    if want_p1:
        in_specs.append(pl.BlockSpec((H, H), lambda i: (0, 0)))
        args.append(wms_next)
        out_specs = [out_specs, pl.BlockSpec((RB, H), lambda i: (i, 0))]
        out_shape = [out_shape, jax.ShapeDtypeStruct((n, H), jnp.float32)]
    return pl.pallas_call(
        body,
        grid=(n // RB,),
        in_specs=in_specs,
        out_specs=out_specs,
        out_shape=out_shape,
        interpret=interpret,
    )(*args)


def _out_body(h_ref, w_ref, out_ref):
    out_ref[...] = jnp.dot(h_ref[...], w_ref[...],
                           preferred_element_type=jnp.float32)


def _out_call(h_g, W_out, RB=512, interpret=False):
    return pl.pallas_call(
        _out_body,
        grid=(NG // RB,),
        in_specs=[
            pl.BlockSpec((RB, H), lambda i: (i, 0)),
            pl.BlockSpec((H, CODE), lambda i: (0, 0)),
        ],
        out_specs=pl.BlockSpec((RB, CODE), lambda i: (i, 0)),
        out_shape=jax.ShapeDtypeStruct((NG, CODE), jnp.float32),
        interpret=interpret,
    )(h_g, W_out)


_AA_CHUNKS = NA * KA // _ROWS_PER_GATHER


def _sc_gather_body(src_hbm, table_hbm, outa_hbm, outg_hbm, idx_v,
                    buf0, buf1, sem0, sem1):
    c = lax.axis_index("c")
    s = lax.axis_index("s")
    wid = s * 2 + c
    nblk = _GATHERS_PER_WORKER
    pltpu.sync_copy(src_hbm.at[wid], idx_v)
    bufs = (buf0, buf1)
    sems = (sem0, sem1)
    copies = [None] * nblk
    copies[0] = pltpu.async_copy(table_hbm.at[idx_v.at[0]], bufs[0], sems[0])
    for j in range(nblk):
        copies[j].wait()
        if j + 1 < nblk:
            copies[j + 1] = pltpu.async_copy(
                table_hbm.at[idx_v.at[j + 1]], bufs[(j + 1) % 2],
                sems[(j + 1) % 2])
        chunk = wid + _SC_WORKERS * j
        if j < _AA_CHUNKS // _SC_WORKERS:
            pltpu.sync_copy(
                bufs[j % 2],
                outa_hbm.at[pl.ds(chunk * _ROWS_PER_GATHER,
                                  _ROWS_PER_GATHER)])
        else:
            pltpu.sync_copy(
                bufs[j % 2],
                outg_hbm.at[pl.ds((chunk - _AA_CHUNKS) * _ROWS_PER_GATHER,
                                  _ROWS_PER_GATHER)])


def _sc_gather(src3, table):
    D = table.shape[1]
    mesh = plsc.VectorSubcoreMesh(core_axis_name="c", subcore_axis_name="s")
    params = (pltpu.CompilerParams(use_tc_tiling_on_sc=False)
              if D < 128 else None)
    fn = pl.kernel(
        _sc_gather_body,
        out_type=[
            jax.ShapeDtypeStruct((NA * KA, D), jnp.float32),
            jax.ShapeDtypeStruct((NG * KG, D), jnp.float32),
        ],
        mesh=mesh,
        compiler_params=params,
        scratch_types=[
            pltpu.VMEM((_GATHERS_PER_WORKER, _ROWS_PER_GATHER), jnp.int32),
            pltpu.VMEM((_ROWS_PER_GATHER, D), jnp.float32),
            pltpu.VMEM((_ROWS_PER_GATHER, D), jnp.float32),
            pltpu.SemaphoreType.DMA,
            pltpu.SemaphoreType.DMA,
        ],
    )
    return fn(src3, table)


def _grid_xyz():
    lin = (jnp.arange(GRID, dtype=jnp.float32) - (GRID - 1) / 2.0) * SPACING
    gx, gy, gz = jnp.meshgrid(lin, lin, lin, indexing='ij')
    return jnp.stack([gx.ravel(), gy.ravel(), gz.ravel()], axis=-1)


def _pipeline(pos, atom_types, batch, W_embed, W_msg, W_upd, W_out,
              gather_fn, interpret=False):
    f32 = jnp.float32
    i32 = jnp.int32
    batch = batch.astype(i32)
    atom_types = atom_types.astype(i32)

    grid_flat = jnp.tile(_grid_xyz(), (NB, 1))
    grid_batch = jnp.repeat(jnp.arange(NB, dtype=i32), NGRID)

    pos8 = jnp.pad(pos, ((0, 0), (0, 5)))
    posT = jnp.pad(pos.T, ((0, 5), (0, 0)))
    batT = jnp.broadcast_to(batch[None, :], (8, NA))
    bat8 = jnp.broadcast_to(batch[:, None], (NA, 8))
    gridT = jnp.pad(grid_flat.T, ((0, 5), (0, 0)))
    batGT = jnp.broadcast_to(grid_batch[None, :], (8, NG))
    types8 = jnp.broadcast_to(atom_types[:, None], (NA, 8))

    nbrTA, shTA, attrTA = _knn_call(posT, batT, pos8, bat8, posT, KA, True,
                                    256, interpret=interpret)
    nbrTG, shTG, attrTG = _knn_call(gridT, batGT, pos8, bat8, posT, KG,
                                    False, 256, interpret=interpret)
    nbrA = nbrTA.T
    nbrG = nbrTG.T
    shA = shTA.transpose(2, 0, 1)
    shG = shTG.transpose(2, 0, 1)
    attrA = attrTA.T
    attrG = attrTG.T

    src_chunks = jnp.concatenate(
        [nbrA.reshape(-1), nbrG.reshape(-1)]).reshape(
            _GATHERS_PER_WORKER, _SC_WORKERS, _ROWS_PER_GATHER)
    src3 = src_chunks.transpose(1, 0, 2)

    h_a, P1 = _embed_call(types8, W_embed, W_msg[0, :H], interpret=interpret)
    h_g = jnp.zeros((NG, H), f32)

    for l in range(NL):
        wm_d = W_msg[l, H:2 * H]
        we = jnp.zeros((16, H), f32).at[:9].set(W_msg[l, 2 * H:])
        wu_h = W_upd[l, :H]
        wu_a = W_upd[l, H:2 * H]
        wu_n = jnp.zeros((16, H), f32).at[:9].set(W_upd[l, 2 * H:])

        GA, GG = gather_fn(src3, P1)
        Ga = GA.reshape(NA, KA, H)
        Gg = GG.reshape(NG, KG, H)
        h_a, P1 = _msg_call(Ga, shA, h_a, attrA, wm_d, we, wu_h, wu_a, wu_n,
                            KA, 512, wms_next=W_msg[(l + 1) % NL, :H],
                            interpret=interpret)
        h_g = _msg_call(Gg, shG, h_g, attrG, wm_d, we, wu_h, wu_a, wu_n,
                        KG, 512, interpret=interpret)

    out = _out_call(h_g, W_out, interpret=interpret)
    return out.reshape(NB, NGRID, CODE)


def kernel(pos, atom_types, batch, W_embed, W_msg, W_upd, W_out):
    return _pipeline(pos, atom_types, batch, W_embed, W_msg, W_upd, W_out,
                     _sc_gather, interpret=False)


# speedup vs baseline: 1.2426x; 1.1769x over previous
import functools

import jax
import jax.numpy as jnp
from jax import lax
from jax.experimental import pallas as pl
from jax.experimental.pallas import tpu as pltpu
from jax.experimental.pallas import tpu_sc as plsc

NA = 4096
NB = 4
GRID = 8
NGRID = GRID ** 3
NG = NB * NGRID
NT = 16
CODE = 32
H = 128
NL = 4
KA = 8
KG = 24
CUTOFF = 8.0
SPACING = 1.5

_C0 = 0.28209479177387814
_C1 = 0.4886025119029199
_C2 = 1.0925484305920792
_C2M0 = 0.31539156525252005
_C2P2 = 0.5462742152960396

EDGES = NA * KA + NG * KG
_SC_WORKERS = 32
_ROWS_PER_GATHER = 128
_GATHERS_PER_WORKER = EDGES // (_SC_WORKERS * _ROWS_PER_GATHER)


def _knn_body(K, diag, RB, C, dstT_ref, dstbT_ref, pos8_ref, bat8_ref,
              posT_ref, nbrT_ref, shT_ref, attrT_ref):
    b = pl.program_id(0)
    pxc = pos8_ref[:, 0:1]
    pyc = pos8_ref[:, 1:2]
    pzc = pos8_ref[:, 2:3]
    batc = bat8_ref[:, 0:1]
    dstx = dstT_ref[0:1, :]
    dsty = dstT_ref[1:2, :]
    dstz = dstT_ref[2:3, :]
    dx = pxc - dstx
    dy = pyc - dsty
    dz = pzc - dstz
    d2 = dx * dx + dy * dy + dz * dz
    inf = jnp.float32(jnp.inf)
    same = batc == dstbT_ref[0:1, :]
    d2 = jnp.where(same, d2, inf)
    rows = lax.broadcasted_iota(jnp.int32, (C, RB), 0)
    if diag:
        cols = lax.broadcasted_iota(jnp.int32, (C, RB), 1) + b * RB
        d2 = jnp.where(rows == cols, inf, d2)
    posT = posT_ref[...]

    def body(k, carry):
        d2, sums, cnt = carry
        idx = jnp.argmin(d2, axis=0).astype(jnp.int32).reshape(1, RB)
        oh = rows == idx
        d2 = jnp.where(oh, inf, d2)
        nbrT_ref[pl.ds(k, 1), :] = idx
        selp = jnp.dot(posT, oh.astype(jnp.float32),
                       preferred_element_type=jnp.float32)
        rx = selp[0:1, :] - dstx
        ry = selp[1:2, :] - dsty
        rz = selp[2:3, :] - dstz
        ex = rx + 1e-12
        ey = ry + 1e-12
        ez = rz + 1e-12
        dist = jnp.sqrt(ex * ex + ey * ey + ez * ez)
        valid = (dist <= CUTOFF).astype(jnp.float32)
        rn = jnp.sqrt(rx * rx + ry * ry + rz * rz)
        inv = 1.0 / (rn + 1e-12)
        x = rx * inv
        y = ry * inv
        z = rz * inv
        sh = jnp.concatenate([
            jnp.full_like(x, _C0),
            _C1 * y, _C1 * z, _C1 * x,
            _C2 * x * y, _C2 * y * z, _C2M0 * (3.0 * z * z - 1.0),
            _C2 * x * z, _C2P2 * (x * x - y * y),
        ], axis=0) * valid
        shv = jnp.concatenate(
            [sh, valid, jnp.zeros((6, RB), jnp.float32)], axis=0)
        shT_ref[pl.ds(k, 1), :, :] = shv[None]
        return d2, sums + shv, cnt + valid

    carry = (d2, jnp.zeros((16, RB), jnp.float32),
             jnp.zeros((1, RB), jnp.float32))
    d2, sums, cnt = lax.fori_loop(0, K, body, carry)
    mean = sums / jnp.maximum(cnt, 1.0)
    attrT_ref[...] = jnp.concatenate(
        [jnp.ones((1, RB), jnp.float32), mean[1:9, :],
         jnp.zeros((7, RB), jnp.float32)], axis=0)


def _knn_call(dstT, dstbT, pos8, bat8, posT, K, diag, RB, interpret=False):
    n = dstT.shape[1]
    C = pos8.shape[0]
    grid = (n // RB,)
    body = functools.partial(_knn_body, K, diag, RB, C)
    return pl.pallas_call(
        body,
        grid=grid,
        in_specs=[
            pl.BlockSpec((8, RB), lambda i: (0, i)),
            pl.BlockSpec((8, RB), lambda i: (0, i)),
            pl.BlockSpec((C, 8), lambda i: (0, 0)),
            pl.BlockSpec((C, 8), lambda i: (0, 0)),
            pl.BlockSpec((8, C), lambda i: (0, 0)),
        ],
        out_specs=[
            pl.BlockSpec((K, RB), lambda i: (0, i)),
            pl.BlockSpec((K, 16, RB), lambda i: (0, 0, i)),
            pl.BlockSpec((16, RB), lambda i: (0, i)),
        ],
        out_shape=[
            jax.ShapeDtypeStruct((K, n), jnp.int32),
            jax.ShapeDtypeStruct((K, 16, n), jnp.float32),
            jax.ShapeDtypeStruct((16, n), jnp.float32),
        ],
        interpret=interpret,
    )(dstT, dstbT, pos8, bat8, posT)


def _embed_body(types_ref, wemb_ref, wms_ref, out_ref, p1_ref):
    t = types_ref[:, 0:1]
    ids = lax.broadcasted_iota(jnp.int32, (1, NT), 1)
    oh = (t == ids).astype(jnp.float32)
    h = jnp.dot(oh, wemb_ref[...], preferred_element_type=jnp.float32)
    out_ref[...] = h
    p1_ref[...] = jnp.dot(h, wms_ref[...], preferred_element_type=jnp.float32)


def _embed_call(types8, W_embed, wm_s0, RB=512, interpret=False):
    return pl.pallas_call(
        _embed_body,
        grid=(NA // RB,),
        in_specs=[
            pl.BlockSpec((RB, 8), lambda i: (i, 0)),
            pl.BlockSpec((NT, H), lambda i: (0, 0)),
            pl.BlockSpec((H, H), lambda i: (0, 0)),
        ],
        out_specs=[
            pl.BlockSpec((RB, H), lambda i: (i, 0)),
            pl.BlockSpec((RB, H), lambda i: (i, 0)),
        ],
        out_shape=[
            jax.ShapeDtypeStruct((NA, H), jnp.float32),
            jax.ShapeDtypeStruct((NA, H), jnp.float32),
        ],
        interpret=interpret,
    )(types8, W_embed, wm_s0)


def _msg_body(K, RB, want_p1, *refs):
    if want_p1:
        (g3_ref, sh3_ref, h_ref, attr_ref, wmd_ref, we_ref,
         wuh_ref, wua_ref, wun_ref, wmsn_ref, out_ref, p1_ref) = refs
    else:
        (g3_ref, sh3_ref, h_ref, attr_ref, wmd_ref, we_ref,
         wuh_ref, wua_ref, wun_ref, out_ref) = refs
    hb = h_ref[...]
    p2 = jnp.dot(hb, wmd_ref[...], preferred_element_type=jnp.float32)
    agg = jnp.zeros((RB, H), jnp.float32)
    for k in range(K):
        shk = sh3_ref[:, k, :]
        ep = jnp.dot(shk, we_ref[...], preferred_element_type=jnp.float32)
        gk = g3_ref[:, k, :]
        mk = jax.nn.silu(gk + p2 + ep) * shk[:, 9:10]
        agg = agg + mk
    u = (jnp.dot(hb, wuh_ref[...], preferred_element_type=jnp.float32)
         + jnp.dot(agg, wua_ref[...], preferred_element_type=jnp.float32)
         + jnp.dot(attr_ref[...], wun_ref[...],
                   preferred_element_type=jnp.float32))
    hn = hb + jax.nn.silu(u)
    out_ref[...] = hn
    if want_p1:
        p1_ref[...] = jnp.dot(hn, wmsn_ref[...],
                              preferred_element_type=jnp.float32)


def _msg_call(g3, sh3, h, attr, wmd, we, wuh, wua, wun, K, RB,
              wms_next=None, interpret=False):
    n = h.shape[0]
    want_p1 = wms_next is not None
    body = functools.partial(_msg_body, K, RB, want_p1)
    in_specs = [
        pl.BlockSpec((RB, K, H), lambda i: (i, 0, 0)),
        pl.BlockSpec((RB, K, 16), lambda i: (i, 0, 0)),
        pl.BlockSpec((RB, H), lambda i: (i, 0)),
        pl.BlockSpec((RB, 16), lambda i: (i, 0)),
        pl.BlockSpec((H, H), lambda i: (0, 0)),
        pl.BlockSpec((16, H), lambda i: (0, 0)),
        pl.BlockSpec((H, H), lambda i: (0, 0)),
        pl.BlockSpec((H, H), lambda i: (0, 0)),
        pl.BlockSpec((16, H), lambda i: (0, 0)),
    ]
    args = [g3, sh3, h, attr, wmd, we, wuh, wua, wun]
    out_specs = pl.BlockSpec((RB, H), lambda i: (i, 0))
    out_shape = jax.ShapeDtypeStruct((n, H), jnp.float32)
    if want_p1:
        in_specs.append(pl.BlockSpec((H, H), lambda i: (0, 0)))
        args.append(wms_next)
        out_specs = [out_specs, pl.BlockSpec((RB, H), lambda i: (i, 0))]
        out_shape = [out_shape, jax.ShapeDtypeStruct((n, H), jnp.float32)]
    return pl.pallas_call(
        body,
        grid=(n // RB,),
        in_specs=in_specs,
        out_specs=out_specs,
        out_shape=out_shape,
        interpret=interpret,
    )(*args)


def _out_body(h_ref, w_ref, out_ref):
    out_ref[...] = jnp.dot(h_ref[...], w_ref[...],
                           preferred_element_type=jnp.float32)


def _out_call(h_g, W_out, RB=512, interpret=False):
    return pl.pallas_call(
        _out_body,
        grid=(NG // RB,),
        in_specs=[
            pl.BlockSpec((RB, H), lambda i: (i, 0)),
            pl.BlockSpec((H, CODE), lambda i: (0, 0)),
        ],
        out_specs=pl.BlockSpec((RB, CODE), lambda i: (i, 0)),
        out_shape=jax.ShapeDtypeStruct((NG, CODE), jnp.float32),
        interpret=interpret,
    )(h_g, W_out)


_AA_CHUNKS = NA * KA // _ROWS_PER_GATHER


def _sc_gather_body(src_hbm, table_hbm, outa_hbm, outg_hbm, idx_v,
                    buf0, buf1, sem0, sem1):
    c = lax.axis_index("c")
    s = lax.axis_index("s")
    wid = s * 2 + c
    nblk = _GATHERS_PER_WORKER
    pltpu.sync_copy(src_hbm.at[wid], idx_v)
    bufs = (buf0, buf1)
    sems = (sem0, sem1)
    copies = [None] * nblk
    copies[0] = pltpu.async_copy(table_hbm.at[idx_v.at[0]], bufs[0], sems[0])
    for j in range(nblk):
        copies[j].wait()
        if j + 1 < nblk:
            copies[j + 1] = pltpu.async_copy(
                table_hbm.at[idx_v.at[j + 1]], bufs[(j + 1) % 2],
                sems[(j + 1) % 2])
        chunk = wid + _SC_WORKERS * j
        if j < _AA_CHUNKS // _SC_WORKERS:
            pltpu.sync_copy(
                bufs[j % 2],
                outa_hbm.at[pl.ds(chunk * _ROWS_PER_GATHER,
                                  _ROWS_PER_GATHER)])
        else:
            pltpu.sync_copy(
                bufs[j % 2],
                outg_hbm.at[pl.ds((chunk - _AA_CHUNKS) * _ROWS_PER_GATHER,
                                  _ROWS_PER_GATHER)])


def _sc_gather(src3, table):
    D = table.shape[1]
    mesh = plsc.VectorSubcoreMesh(core_axis_name="c", subcore_axis_name="s")
    params = (pltpu.CompilerParams(use_tc_tiling_on_sc=False)
              if D < 128 else None)
    fn = pl.kernel(
        _sc_gather_body,
        out_type=[
            jax.ShapeDtypeStruct((NA * KA, D), jnp.float32),
            jax.ShapeDtypeStruct((NG * KG, D), jnp.float32),
        ],
        mesh=mesh,
        compiler_params=params,
        scratch_types=[
            pltpu.VMEM((_GATHERS_PER_WORKER, _ROWS_PER_GATHER), jnp.int32),
            pltpu.VMEM((_ROWS_PER_GATHER, D), jnp.float32),
            pltpu.VMEM((_ROWS_PER_GATHER, D), jnp.float32),
            pltpu.SemaphoreType.DMA,
            pltpu.SemaphoreType.DMA,
        ],
    )
    return fn(src3, table)


def _grid_xyz():
    lin = (jnp.arange(GRID, dtype=jnp.float32) - (GRID - 1) / 2.0) * SPACING
    gx, gy, gz = jnp.meshgrid(lin, lin, lin, indexing='ij')
    return jnp.stack([gx.ravel(), gy.ravel(), gz.ravel()], axis=-1)


def _pipeline(pos, atom_types, batch, W_embed, W_msg, W_upd, W_out,
              gather_fn, interpret=False):
    f32 = jnp.float32
    i32 = jnp.int32
    batch = batch.astype(i32)
    atom_types = atom_types.astype(i32)

    grid_flat = jnp.tile(_grid_xyz(), (NB, 1))
    grid_batch = jnp.repeat(jnp.arange(NB, dtype=i32), NGRID)

    pos8 = jnp.pad(pos, ((0, 0), (0, 5)))
    posT = jnp.pad(pos.T, ((0, 5), (0, 0)))
    batT = jnp.broadcast_to(batch[None, :], (8, NA))
    bat8 = jnp.broadcast_to(batch[:, None], (NA, 8))
    gridT = jnp.pad(grid_flat.T, ((0, 5), (0, 0)))
    batGT = jnp.broadcast_to(grid_batch[None, :], (8, NG))
    types8 = jnp.broadcast_to(atom_types[:, None], (NA, 8))

    nbrTA, shTA, attrTA = _knn_call(posT, batT, pos8, bat8, posT, KA, True,
                                    256, interpret=interpret)
    nbrTG, shTG, attrTG = _knn_call(gridT, batGT, pos8, bat8, posT, KG,
                                    False, 256, interpret=interpret)
    nbrA = nbrTA.T
    nbrG = nbrTG.T
    shA = shTA.transpose(2, 0, 1)
    shG = shTG.transpose(2, 0, 1)
    attrA = attrTA.T
    attrG = attrTG.T

    src_chunks = jnp.concatenate(
        [nbrA.reshape(-1), nbrG.reshape(-1)]).reshape(
            _GATHERS_PER_WORKER, _SC_WORKERS, _ROWS_PER_GATHER)
    src3 = src_chunks.transpose(1, 0, 2)

    h_a, P1 = _embed_call(types8, W_embed, W_msg[0, :H], interpret=interpret)
    h_g = jnp.zeros((NG, H), f32)

    for l in range(NL):
        wm_d = W_msg[l, H:2 * H]
        we = jnp.zeros((16, H), f32).at[:9].set(W_msg[l, 2 * H:])
        wu_h = W_upd[l, :H]
        wu_a = W_upd[l, H:2 * H]
        wu_n = jnp.zeros((16, H), f32).at[:9].set(W_upd[l, 2 * H:])

        GA, GG = gather_fn(src3, P1)
        Ga = GA.reshape(NA, KA, H)
        Gg = GG.reshape(NG, KG, H)
        h_a, P1 = _msg_call(Ga, shA, h_a, attrA, wm_d, we, wu_h, wu_a, wu_n,
                            KA, 512, wms_next=W_msg[(l + 1) % NL, :H],
                            interpret=interpret)
        h_g = _msg_call(Gg, shG, h_g, attrG, wm_d, we, wu_h, wu_a, wu_n,
                        KG, 512, interpret=interpret)

    out = _out_call(h_g, W_out, interpret=interpret)
    return out.reshape(NB, NGRID, CODE)


def kernel(pos, atom_types, batch, W_embed, W_msg, W_upd, W_out):
    return _pipeline(pos, atom_types, batch, W_embed, W_msg, W_upd, W_out,
                     _sc_gather, interpret=False)


# speedup vs baseline: 1.3074x; 1.0521x over previous
import functools

import jax
import jax.numpy as jnp
from jax import lax
from jax.experimental import pallas as pl
from jax.experimental.pallas import tpu as pltpu
from jax.experimental.pallas import tpu_sc as plsc

NA = 4096
NB = 4
GRID = 8
NGRID = GRID ** 3
NG = NB * NGRID
NT = 16
CODE = 32
H = 128
NL = 4
KA = 8
KG = 24
CUTOFF = 8.0
SPACING = 1.5

_C0 = 0.28209479177387814
_C1 = 0.4886025119029199
_C2 = 1.0925484305920792
_C2M0 = 0.31539156525252005
_C2P2 = 0.5462742152960396

EDGES = NA * KA + NG * KG
_SC_WORKERS = 32
_ROWS_PER_GATHER = 128
_GATHERS_PER_WORKER = EDGES // (_SC_WORKERS * _ROWS_PER_GATHER)


def _knn_body(K, diag, RB, C, dstT_ref, dstbT_ref, pos8_ref, bat8_ref,
              posT_ref, nbrT_ref, shT_ref, attrT_ref):
    b = pl.program_id(0)
    pxc = pos8_ref[:, 0:1]
    pyc = pos8_ref[:, 1:2]
    pzc = pos8_ref[:, 2:3]
    batc = bat8_ref[:, 0:1]
    dstx = dstT_ref[0:1, :]
    dsty = dstT_ref[1:2, :]
    dstz = dstT_ref[2:3, :]
    dx = pxc - dstx
    dy = pyc - dsty
    dz = pzc - dstz
    d2 = dx * dx + dy * dy + dz * dz
    inf = jnp.float32(jnp.inf)
    same = batc == dstbT_ref[0:1, :]
    d2 = jnp.where(same, d2, inf)
    rows = lax.broadcasted_iota(jnp.int32, (C, RB), 0)
    if diag:
        cols = lax.broadcasted_iota(jnp.int32, (C, RB), 1) + b * RB
        d2 = jnp.where(rows == cols, inf, d2)
    posT = posT_ref[...]

    def body(k, carry):
        d2, sums, cnt = carry
        idx = jnp.argmin(d2, axis=0).astype(jnp.int32).reshape(1, RB)
        oh = rows == idx
        d2 = jnp.where(oh, inf, d2)
        nbrT_ref[pl.ds(k, 1), :] = idx
        selp = jnp.dot(posT, oh.astype(jnp.float32),
                       preferred_element_type=jnp.float32)
        rx = selp[0:1, :] - dstx
        ry = selp[1:2, :] - dsty
        rz = selp[2:3, :] - dstz
        ex = rx + 1e-12
        ey = ry + 1e-12
        ez = rz + 1e-12
        dist = jnp.sqrt(ex * ex + ey * ey + ez * ez)
        valid = (dist <= CUTOFF).astype(jnp.float32)
        rn = jnp.sqrt(rx * rx + ry * ry + rz * rz)
        inv = 1.0 / (rn + 1e-12)
        x = rx * inv
        y = ry * inv
        z = rz * inv
        sh = jnp.concatenate([
            jnp.full_like(x, _C0),
            _C1 * y, _C1 * z, _C1 * x,
            _C2 * x * y, _C2 * y * z, _C2M0 * (3.0 * z * z - 1.0),
            _C2 * x * z, _C2P2 * (x * x - y * y),
        ], axis=0) * valid
        shv = jnp.concatenate(
            [sh, valid, jnp.zeros((6, RB), jnp.float32)], axis=0)
        shT_ref[pl.ds(k, 1), :, :] = shv[None]
        return d2, sums + shv, cnt + valid

    carry = (d2, jnp.zeros((16, RB), jnp.float32),
             jnp.zeros((1, RB), jnp.float32))
    d2, sums, cnt = lax.fori_loop(0, K, body, carry)
    mean = sums / jnp.maximum(cnt, 1.0)
    attrT_ref[...] = jnp.concatenate(
        [jnp.ones((1, RB), jnp.float32), mean[1:9, :],
         jnp.zeros((7, RB), jnp.float32)], axis=0)


def _knn_call(dstT, dstbT, pos8, bat8, posT, K, diag, RB, interpret=False):
    n = dstT.shape[1]
    C = pos8.shape[0]
    grid = (n // RB,)
    body = functools.partial(_knn_body, K, diag, RB, C)
    return pl.pallas_call(
        body,
        grid=grid,
        in_specs=[
            pl.BlockSpec((8, RB), lambda i: (0, i)),
            pl.BlockSpec((8, RB), lambda i: (0, i)),
            pl.BlockSpec((C, 8), lambda i: (0, 0)),
            pl.BlockSpec((C, 8), lambda i: (0, 0)),
            pl.BlockSpec((8, C), lambda i: (0, 0)),
        ],
        out_specs=[
            pl.BlockSpec((K, RB), lambda i: (0, i)),
            pl.BlockSpec((K, 16, RB), lambda i: (0, 0, i)),
            pl.BlockSpec((16, RB), lambda i: (0, i)),
        ],
        out_shape=[
            jax.ShapeDtypeStruct((K, n), jnp.int32),
            jax.ShapeDtypeStruct((K, 16, n), jnp.float32),
            jax.ShapeDtypeStruct((16, n), jnp.float32),
        ],
        interpret=interpret,
    )(dstT, dstbT, pos8, bat8, posT)


def _embed_body(types_ref, wemb_ref, wms_ref, out_ref, p1_ref):
    t = types_ref[:, 0:1]
    ids = lax.broadcasted_iota(jnp.int32, (1, NT), 1)
    oh = (t == ids).astype(jnp.float32)
    h = jnp.dot(oh, wemb_ref[...], preferred_element_type=jnp.float32)
    out_ref[...] = h
    p1_ref[...] = jnp.dot(h, wms_ref[...], preferred_element_type=jnp.float32)


def _embed_call(types8, W_embed, wm_s0, RB=512, interpret=False):
    return pl.pallas_call(
        _embed_body,
        grid=(NA // RB,),
        in_specs=[
            pl.BlockSpec((RB, 8), lambda i: (i, 0)),
            pl.BlockSpec((NT, H), lambda i: (0, 0)),
            pl.BlockSpec((H, H), lambda i: (0, 0)),
        ],
        out_specs=[
            pl.BlockSpec((RB, H), lambda i: (i, 0)),
            pl.BlockSpec((RB, H), lambda i: (i, 0)),
        ],
        out_shape=[
            jax.ShapeDtypeStruct((NA, H), jnp.float32),
            jax.ShapeDtypeStruct((NA, H), jnp.float32),
        ],
        interpret=interpret,
    )(types8, W_embed, wm_s0)


def _msg_body(K, RB, want_p1, *refs):
    if want_p1:
        (g3_ref, sh3_ref, h_ref, attr_ref, wmd_ref, we_ref,
         wuh_ref, wua_ref, wun_ref, wmsn_ref, out_ref, p1_ref) = refs
    else:
        (g3_ref, sh3_ref, h_ref, attr_ref, wmd_ref, we_ref,
         wuh_ref, wua_ref, wun_ref, out_ref) = refs
    hb = h_ref[...]
    p2 = jnp.dot(hb, wmd_ref[...], preferred_element_type=jnp.float32)
    agg = jnp.zeros((RB, H), jnp.float32)
    for k in range(K):
        shk = sh3_ref[:, k, :]
        ep = jnp.dot(shk, we_ref[...], preferred_element_type=jnp.float32)
        gk = g3_ref[:, k, :]
        mk = jax.nn.silu(gk + p2 + ep) * shk[:, 9:10]
        agg = agg + mk
    u = (jnp.dot(hb, wuh_ref[...], preferred_element_type=jnp.float32)
         + jnp.dot(agg, wua_ref[...], preferred_element_type=jnp.float32)
         + jnp.dot(attr_ref[...], wun_ref[...],
                   preferred_element_type=jnp.float32))
    hn = hb + jax.nn.silu(u)
    out_ref[...] = hn
    if want_p1:
        p1_ref[...] = jnp.dot(hn, wmsn_ref[...],
                              preferred_element_type=jnp.float32)


def _msg_call(g3, sh3, h, attr, wmd, we, wuh, wua, wun, K, RB,
              wms_next=None, interpret=False):
    n = h.shape[0]
    want_p1 = wms_next is not None
    body = functools.partial(_msg_body, K, RB, want_p1)
    in_specs = [
        pl.BlockSpec((RB, K, H), lambda i: (i, 0, 0)),
        pl.BlockSpec((RB, K, 16), lambda i: (i, 0, 0)),
        pl.BlockSpec((RB, H), lambda i: (i, 0)),
        pl.BlockSpec((RB, 16), lambda i: (i, 0)),
        pl.BlockSpec((H, H), lambda i: (0, 0)),
        pl.BlockSpec((16, H), lambda i: (0, 0)),
        pl.BlockSpec((H, H), lambda i: (0, 0)),
        pl.BlockSpec((H, H), lambda i: (0, 0)),
        pl.BlockSpec((16, H), lambda i: (0, 0)),
    ]
    args = [g3, sh3, h, attr, wmd, we, wuh, wua, wun]
    out_specs = pl.BlockSpec((RB, H), lambda i: (i, 0))
    out_shape = jax.ShapeDtypeStruct((n, H), jnp.float32)
    if want_p1:
        in_specs.append(pl.BlockSpec((H, H), lambda i: (0, 0)))
        args.append(wms_next)
        out_specs = [out_specs, pl.BlockSpec((RB, H), lambda i: (i, 0))]
        out_shape = [out_shape, jax.ShapeDtypeStruct((n, H), jnp.float32)]
    return pl.pallas_call(
        body,
        grid=(n // RB,),
        in_specs=in_specs,
        out_specs=out_specs,
        out_shape=out_shape,
        interpret=interpret,
    )(*args)


def _out_body(h_ref, w_ref, out_ref):
    out_ref[...] = jnp.dot(h_ref[...], w_ref[...],
                           preferred_element_type=jnp.float32)


def _out_call(h_g, W_out, RB=512, interpret=False):
    return pl.pallas_call(
        _out_body,
        grid=(NG // RB,),
        in_specs=[
            pl.BlockSpec((RB, H), lambda i: (i, 0)),
            pl.BlockSpec((H, CODE), lambda i: (0, 0)),
        ],
        out_specs=pl.BlockSpec((RB, CODE), lambda i: (i, 0)),
        out_shape=jax.ShapeDtypeStruct((NG, CODE), jnp.float32),
        interpret=interpret,
    )(h_g, W_out)


_AA_CHUNKS = NA * KA // _ROWS_PER_GATHER


def _sc_gather_body(src_hbm, table_hbm, outa_hbm, outg_hbm, idx_v,
                    buf0, buf1, sem0, sem1):
    c = lax.axis_index("c")
    s = lax.axis_index("s")
    wid = s * 2 + c
    nblk = _GATHERS_PER_WORKER
    pltpu.sync_copy(src_hbm.at[wid], idx_v)
    bufs = (buf0, buf1)
    sems = (sem0, sem1)
    copies = [None] * nblk
    copies[0] = pltpu.async_copy(table_hbm.at[idx_v.at[0]], bufs[0], sems[0])
    for j in range(nblk):
        copies[j].wait()
        if j + 1 < nblk:
            copies[j + 1] = pltpu.async_copy(
                table_hbm.at[idx_v.at[j + 1]], bufs[(j + 1) % 2],
                sems[(j + 1) % 2])
        chunk = wid + _SC_WORKERS * j
        if j < _AA_CHUNKS // _SC_WORKERS:
            pltpu.sync_copy(
                bufs[j % 2],
                outa_hbm.at[pl.ds(chunk * _ROWS_PER_GATHER,
                                  _ROWS_PER_GATHER)])
        else:
            pltpu.sync_copy(
                bufs[j % 2],
                outg_hbm.at[pl.ds((chunk - _AA_CHUNKS) * _ROWS_PER_GATHER,
                                  _ROWS_PER_GATHER)])


def _sc_gather(src3, table):
    D = table.shape[1]
    mesh = plsc.VectorSubcoreMesh(core_axis_name="c", subcore_axis_name="s")
    params = (pltpu.CompilerParams(use_tc_tiling_on_sc=False)
              if D < 128 else None)
    fn = pl.kernel(
        _sc_gather_body,
        out_type=[
            jax.ShapeDtypeStruct((NA * KA, D), jnp.float32),
            jax.ShapeDtypeStruct((NG * KG, D), jnp.float32),
        ],
        mesh=mesh,
        compiler_params=params,
        scratch_types=[
            pltpu.VMEM((_GATHERS_PER_WORKER, _ROWS_PER_GATHER), jnp.int32),
            pltpu.VMEM((_ROWS_PER_GATHER, D), jnp.float32),
            pltpu.VMEM((_ROWS_PER_GATHER, D), jnp.float32),
            pltpu.SemaphoreType.DMA,
            pltpu.SemaphoreType.DMA,
        ],
    )
    return fn(src3, table)


def _grid_xyz():
    lin = (jnp.arange(GRID, dtype=jnp.float32) - (GRID - 1) / 2.0) * SPACING
    gx, gy, gz = jnp.meshgrid(lin, lin, lin, indexing='ij')
    return jnp.stack([gx.ravel(), gy.ravel(), gz.ravel()], axis=-1)


def _pipeline(pos, atom_types, batch, W_embed, W_msg, W_upd, W_out,
              gather_fn, interpret=False):
    f32 = jnp.float32
    i32 = jnp.int32
    batch = batch.astype(i32)
    atom_types = atom_types.astype(i32)

    grid_flat = jnp.tile(_grid_xyz(), (NB, 1))
    grid_batch = jnp.repeat(jnp.arange(NB, dtype=i32), NGRID)

    pos8 = jnp.pad(pos, ((0, 0), (0, 5)))
    posT = jnp.pad(pos.T, ((0, 5), (0, 0)))
    batT = jnp.broadcast_to(batch[None, :], (8, NA))
    bat8 = jnp.broadcast_to(batch[:, None], (NA, 8))
    gridT = jnp.pad(grid_flat.T, ((0, 5), (0, 0)))
    batGT = jnp.broadcast_to(grid_batch[None, :], (8, NG))
    types8 = jnp.broadcast_to(atom_types[:, None], (NA, 8))

    nbrTA, shTA, attrTA = _knn_call(posT, batT, pos8, bat8, posT, KA, True,
                                    512, interpret=interpret)
    nbrTG, shTG, attrTG = _knn_call(gridT, batGT, pos8, bat8, posT, KG,
                                    False, 512, interpret=interpret)
    nbrA = nbrTA.T
    nbrG = nbrTG.T
    shA = shTA.transpose(2, 0, 1)
    shG = shTG.transpose(2, 0, 1)
    attrA = attrTA.T
    attrG = attrTG.T

    src_chunks = jnp.concatenate(
        [nbrA.reshape(-1), nbrG.reshape(-1)]).reshape(
            _GATHERS_PER_WORKER, _SC_WORKERS, _ROWS_PER_GATHER)
    src3 = src_chunks.transpose(1, 0, 2)

    h_a, P1 = _embed_call(types8, W_embed, W_msg[0, :H], interpret=interpret)
    h_g = jnp.zeros((NG, H), f32)

    for l in range(NL):
        wm_d = W_msg[l, H:2 * H]
        we = jnp.zeros((16, H), f32).at[:9].set(W_msg[l, 2 * H:])
        wu_h = W_upd[l, :H]
        wu_a = W_upd[l, H:2 * H]
        wu_n = jnp.zeros((16, H), f32).at[:9].set(W_upd[l, 2 * H:])

        GA, GG = gather_fn(src3, P1)
        Ga = GA.reshape(NA, KA, H)
        Gg = GG.reshape(NG, KG, H)
        h_a, P1 = _msg_call(Ga, shA, h_a, attrA, wm_d, we, wu_h, wu_a, wu_n,
                            KA, 512, wms_next=W_msg[(l + 1) % NL, :H],
                            interpret=interpret)
        h_g = _msg_call(Gg, shG, h_g, attrG, wm_d, we, wu_h, wu_a, wu_n,
                        KG, 512, interpret=interpret)

    out = _out_call(h_g, W_out, interpret=interpret)
    return out.reshape(NB, NGRID, CODE)


def kernel(pos, atom_types, batch, W_embed, W_msg, W_upd, W_out):
    return _pipeline(pos, atom_types, batch, W_embed, W_msg, W_upd, W_out,
                     _sc_gather, interpret=False)
